# baseline (speedup 1.0000x reference)
import functools

import jax
import jax.numpy as jnp
from jax import lax
from jax.experimental import pallas as pl
from jax.experimental.pallas import tpu as pltpu
from jax.experimental.pallas import tpu_sc as plsc

SIGMA = 2.0
RADIUS = 6
W = 13
L = 100000
D = 64
B = 16384

NUM_WORKERS = 32
NB = B // NUM_WORKERS
NGROUPS = NB // 16
CHB = 64
NCH = NB // CHB
CHROWS = CHB * W


def _tile_body(s_hbm, tok_hbm, tab_hbm, out_hbm,
               s_v, pos_v, tokc_v, rows_v, out_v, sem):
    cid = lax.axis_index("c")
    sid = lax.axis_index("s")
    wid = sid * 2 + cid
    b0 = wid * NB

    pltpu.sync_copy(s_hbm.at[pl.ds(b0, NB)], s_v.at[pl.ds(0, NB)])

    def wgt_body(g, carry):
        sv = s_v[pl.ds(g * 16, 16)]
        base = sv.astype(jnp.int32)
        for j in range(W):
            idx = base + (j - RADIUS)
            pos_v[pl.ds(j * NB + g * 16, 16)] = jnp.clip(idx, 0, L - 1)
        return carry

    lax.fori_loop(0, NGROUPS, wgt_body, 0)

    def chunk_body(c, carry):
        tok_copies = [
            pltpu.async_copy(
                tok_hbm.at[pos_v.at[pl.ds(j * NB + c * CHB, CHB)]],
                tokc_v.at[pl.ds(j * CHB, CHB)], sem)
            for j in range(W)
        ]
        for cp in tok_copies:
            cp.wait()
        def row_batch(rb, carry3):
            r0 = rb * 16
            cps = []
            for k in range(16):
                t = tokc_v[pl.ds(r0 + k, 16)][0]
                cps.append(pltpu.async_copy(
                    tab_hbm.at[pl.ds(t, 1)],
                    rows_v.at[pl.ds(r0 + k, 1)], sem))
            for cp in cps:
                cp.wait()
            return carry3

        lax.fori_loop(0, CHROWS // 16, row_batch, 0)

        def b_body(bl, carry2):
            b = c * CHB + bl
            sval = s_v[pl.ds(b, 16)][0]
            sv = jnp.full((16,), sval, jnp.float32)
            base = sv.astype(jnp.int32)
            basef = base.astype(jnp.float32)
            acc0 = jnp.zeros((16,), jnp.float32)
            acc1 = jnp.zeros((16,), jnp.float32)
            acc2 = jnp.zeros((16,), jnp.float32)
            acc3 = jnp.zeros((16,), jnp.float32)
            wsum = jnp.zeros((16,), jnp.float32)
            for j in range(W):
                idx = base + (j - RADIUS)
                validf = jnp.where((idx >= 0) & (idx < L), 1.0, 0.0)
                z = (basef + float(j - RADIUS) - sv) * (1.0 / SIGMA)
                wj = jnp.exp(-0.5 * z * z) * validf
                wsum = wsum + wj
                r = j * CHB + bl
                acc0 = acc0 + wj * rows_v[r, pl.ds(0, 16)]
                acc1 = acc1 + wj * rows_v[r, pl.ds(16, 16)]
                acc2 = acc2 + wj * rows_v[r, pl.ds(32, 16)]
                acc3 = acc3 + wj * rows_v[r, pl.ds(48, 16)]
            inv = 1.0 / jnp.maximum(wsum, 1e-8)
            out_v[bl, pl.ds(0, 16)] = acc0 * inv
            out_v[bl, pl.ds(16, 16)] = acc1 * inv
            out_v[bl, pl.ds(32, 16)] = acc2 * inv
            out_v[bl, pl.ds(48, 16)] = acc3 * inv
            return carry2

        lax.fori_loop(0, CHB, b_body, 0)
        pltpu.sync_copy(out_v, out_hbm.at[pl.ds(b0 + c * CHB, CHB)])
        return carry

    lax.fori_loop(0, NCH, chunk_body, 0)


@functools.lru_cache(maxsize=1)
def _build():
    @functools.partial(
        pl.kernel,
        out_type=jax.ShapeDtypeStruct((B, D), jnp.float32),
        mesh=plsc.VectorSubcoreMesh(
            core_axis_name="c", subcore_axis_name="s"),
        scratch_types=[
            pltpu.VMEM((NB + 16,), jnp.float32),
            pltpu.VMEM((W * NB,), jnp.int32),
            pltpu.VMEM((CHROWS + 16,), jnp.int32),
            pltpu.VMEM((CHROWS, D), jnp.float32),
            pltpu.VMEM((CHB, D), jnp.float32),
            pltpu.SemaphoreType.DMA,
        ],
    )
    def _gauss_read(s_hbm, tok_hbm, tab_hbm, out_hbm,
                    s_v, pos_v, tokc_v, rows_v, out_v, sem):
        _tile_body(s_hbm, tok_hbm, tab_hbm, out_hbm,
                   s_v, pos_v, tokc_v, rows_v, out_v, sem)

    return _gauss_read


def kernel(s, token_ids, table):
    assert s.shape == (B,) and token_ids.shape == (L,)
    assert table.shape[1] == D
    return _build()(s, token_ids, table)


# speedup vs baseline: 1.1204x; 1.1204x over previous
import functools

import jax
import jax.numpy as jnp
from jax import lax
from jax.experimental import pallas as pl
from jax.experimental.pallas import tpu as pltpu
from jax.experimental.pallas import tpu_sc as plsc

SIGMA = 2.0
RADIUS = 6
W = 13
L = 100000
D = 64
B = 16384

NUM_WORKERS = 32
NB = B // NUM_WORKERS
NGROUPS = NB // 16
CHB = 64
NCH = NB // CHB
CHROWS = CHB * W


def _tile_body(s_hbm, tok_hbm, tab_hbm, out_hbm,
               s_v, pos_v, tokc_v, rows_v, out_v, sem):
    cid = lax.axis_index("c")
    sid = lax.axis_index("s")
    wid = sid * 2 + cid
    b0 = wid * NB

    pltpu.sync_copy(s_hbm.at[pl.ds(b0, NB)], s_v.at[pl.ds(0, NB)])

    def wgt_body(g, carry):
        sv = s_v[pl.ds(g * 16, 16)]
        base = sv.astype(jnp.int32)
        for j in range(W):
            idx = base + (j - RADIUS)
            pos_v[pl.ds(j * NB + g * 16, 16)] = jnp.clip(idx, 0, L - 1)
        return carry

    lax.fori_loop(0, NGROUPS, wgt_body, 0)

    def chunk_body(c, carry):
        tok_copies = [
            pltpu.async_copy(
                tok_hbm.at[pos_v.at[pl.ds(j * NB + c * CHB, CHB)]],
                tokc_v.at[pl.ds(j * CHB, CHB)], sem)
            for j in range(W)
        ]
        for cp in tok_copies:
            cp.wait()
        row_copies = [
            pltpu.async_copy(
                tab_hbm.at[tokc_v.at[pl.ds(j * CHB, CHB)]],
                rows_v.at[pl.ds(j * CHB, CHB)], sem)
            for j in range(W)
        ]
        for cp in row_copies:
            cp.wait()

        def b_body(bl, carry2):
            b = c * CHB + bl
            sval = s_v[pl.ds(b, 16)][0]
            sv = jnp.full((16,), sval, jnp.float32)
            base = sv.astype(jnp.int32)
            basef = base.astype(jnp.float32)
            acc0 = jnp.zeros((16,), jnp.float32)
            acc1 = jnp.zeros((16,), jnp.float32)
            acc2 = jnp.zeros((16,), jnp.float32)
            acc3 = jnp.zeros((16,), jnp.float32)
            wsum = jnp.zeros((16,), jnp.float32)
            for j in range(W):
                idx = base + (j - RADIUS)
                validf = jnp.where((idx >= 0) & (idx < L), 1.0, 0.0)
                z = (basef + float(j - RADIUS) - sv) * (1.0 / SIGMA)
                wj = jnp.exp(-0.5 * z * z) * validf
                wsum = wsum + wj
                r = j * CHB + bl
                acc0 = acc0 + wj * rows_v[r, pl.ds(0, 16)]
                acc1 = acc1 + wj * rows_v[r, pl.ds(16, 16)]
                acc2 = acc2 + wj * rows_v[r, pl.ds(32, 16)]
                acc3 = acc3 + wj * rows_v[r, pl.ds(48, 16)]
            inv = 1.0 / jnp.maximum(wsum, 1e-8)
            out_v[bl, pl.ds(0, 16)] = acc0 * inv
            out_v[bl, pl.ds(16, 16)] = acc1 * inv
            out_v[bl, pl.ds(32, 16)] = acc2 * inv
            out_v[bl, pl.ds(48, 16)] = acc3 * inv
            return carry2

        lax.fori_loop(0, CHB, b_body, 0)
        pltpu.sync_copy(out_v, out_hbm.at[pl.ds(b0 + c * CHB, CHB)])
        return carry

    lax.fori_loop(0, NCH, chunk_body, 0)


@functools.lru_cache(maxsize=1)
def _build():
    @functools.partial(
        pl.kernel,
        out_type=jax.ShapeDtypeStruct((B, D), jnp.float32),
        mesh=plsc.VectorSubcoreMesh(
            core_axis_name="c", subcore_axis_name="s"),
        scratch_types=[
            pltpu.VMEM((NB + 16,), jnp.float32),
            pltpu.VMEM((W * NB,), jnp.int32),
            pltpu.VMEM((CHROWS + 16,), jnp.int32),
            pltpu.VMEM((CHROWS, 2 * D), jnp.float32),
            pltpu.VMEM((CHB, D), jnp.float32),
            pltpu.SemaphoreType.DMA,
        ],
    )
    def _gauss_read(s_hbm, tok_hbm, tab_hbm, out_hbm,
                    s_v, pos_v, tokc_v, rows_v, out_v, sem):
        _tile_body(s_hbm, tok_hbm, tab_hbm, out_hbm,
                   s_v, pos_v, tokc_v, rows_v, out_v, sem)

    return _gauss_read


def kernel(s, token_ids, table):
    assert s.shape == (B,) and token_ids.shape == (L,)
    assert table.shape[1] == D
    tab_pad = jnp.pad(table, ((0, 0), (0, D)))
    return _build()(s, token_ids, tab_pad)
